# baseline (speedup 1.0000x reference)
import functools

import jax
import jax.numpy as jnp
from jax.experimental import pallas as pl
from jax.experimental.pallas import tpu as pltpu

_T = 512
_EPS = 1e-4


def _hnet_body(n_per_batch, x_ref, wq_ref, wk_ref, wres_ref, bres_ref, o_ref,
               bv_s, rr_s, a_s, carry_ref):
    g = pl.program_id(0)
    T = x_ref.shape[1]
    D = x_ref.shape[2]

    sa = jax.lax.rem(g, 2) * T
    sb = jax.lax.rem(g + 1, 2) * T

    bv = bv_s[pl.ds(sb, T), :]
    rrb = rr_s[pl.ds(sb, T), :]
    a = a_s[pl.ds(sb, T), :]

    xb = x_ref[0]
    xh = xb.astype(jnp.bfloat16)
    qq = jnp.dot(xb, wq_ref[...], preferred_element_type=jnp.float32)
    kk = jnp.dot(xb, wk_ref[...], preferred_element_type=jnp.float32)
    rr = jnp.dot(xh, wres_ref[...], preferred_element_type=jnp.float32)
    rr = rr + bres_ref[...]

    row = jax.lax.broadcasted_iota(jnp.int32, (T, 1), 0)
    first_a = jax.lax.rem(g, n_per_batch) == 0
    q_carry = jnp.where(first_a, 0.0, carry_ref[1:2, :])

    q_shift = pltpu.roll(qq, 1, axis=0)

    qn2 = jnp.sum(q_shift * q_shift, axis=1, keepdims=True)
    kn2 = jnp.sum(kk * kk, axis=1, keepdims=True)
    qk = jnp.sum(q_shift * kk, axis=1, keepdims=True)
    qn2_c = jnp.sum(q_carry * q_carry, axis=1, keepdims=True)
    qk_c = jnp.sum(q_carry * kk[0:1, :], axis=1, keepdims=True)
    qn2 = jnp.where(row == 0, qn2_c, qn2)
    qk = jnp.where(row == 0, qk_c, qk)
    cos = qk * jax.lax.rsqrt(jnp.maximum(qn2, 1e-16) * jnp.maximum(kn2, 1e-16))

    p_eff = jnp.where(cos <= 0.0,
                      jnp.minimum(0.5 - 0.5 * cos, 1.0 - _EPS), 0.0)
    p_eff = jnp.where(first_a & (row == 0), 1.0 - _EPS, p_eff)

    bv_s[pl.ds(sa, T), :] = p_eff.astype(jnp.bfloat16) * xh
    rr_s[pl.ds(sa, T), :] = rr.astype(jnp.bfloat16)
    a_s[pl.ds(sa, T), :] = (1.0 - p_eff).astype(jnp.bfloat16)
    carry_ref[1:2, :] = qq[T - 1:T, :]

    first_b = jax.lax.rem(g - 1, n_per_batch) == 0
    z_carry = jnp.where(first_b, 0.0, carry_ref[0:1, :])

    one = jnp.bfloat16(1.0)
    zero = jnp.bfloat16(0.0)

    d = 1
    while d < T:
        pred = row >= d
        am = jnp.where(pred, a, zero)
        bv = am * pltpu.roll(bv, d, axis=0) + bv
        a = a * jnp.where(pred, pltpu.roll(a, d, axis=0), one)
        d *= 2

    o_ref[0] = (rrb + bv).astype(jnp.float32) + a.astype(jnp.float32) * z_carry

    carry_ref[0:1, :] = (bv[T - 1:T, :].astype(jnp.float32)
                         + a[T - 1:T, :].astype(jnp.float32) * z_carry)


def kernel(x, Wq, Wk, Wres, bres):
    B, L, D = x.shape
    T = _T
    N = L // T
    grid = (B * N + 1,)
    last = B * N - 1

    def xmap(g):
        gc = jnp.minimum(g, last)
        return (gc // N, gc % N, 0)

    def omap(g):
        gc = jnp.maximum(g - 1, 0)
        return (gc // N, gc % N, 0)

    out = pl.pallas_call(
        functools.partial(_hnet_body, N),
        grid=grid,
        in_specs=[
            pl.BlockSpec((1, T, D), xmap),
            pl.BlockSpec((D, D), lambda g: (0, 0)),
            pl.BlockSpec((D, D), lambda g: (0, 0)),
            pl.BlockSpec((D, D), lambda g: (0, 0)),
            pl.BlockSpec((1, D), lambda g: (0, 0)),
        ],
        out_specs=pl.BlockSpec((1, T, D), omap),
        out_shape=jax.ShapeDtypeStruct((B, L, D), jnp.float32),
        scratch_shapes=[
            pltpu.VMEM((2 * T, D), jnp.bfloat16),
            pltpu.VMEM((2 * T, D), jnp.bfloat16),
            pltpu.VMEM((2 * T, 1), jnp.bfloat16),
            pltpu.VMEM((2, D), jnp.float32),
        ],
        compiler_params=pltpu.CompilerParams(
            dimension_semantics=("arbitrary",),
        ),
    )(x, Wq, Wk, Wres.astype(jnp.bfloat16), bres.reshape(1, D))
    return out


# speedup vs baseline: 1.0210x; 1.0210x over previous
import functools

import jax
import jax.numpy as jnp
from jax.experimental import pallas as pl
from jax.experimental.pallas import tpu as pltpu

_T = 512
_EPS = 1e-4


def _hnet_body(n_per_batch, x_ref, wq_ref, wk_ref, wres_ref, bres_ref, o_ref,
               bv_s, rr_s, a_s, carry_ref):
    g = pl.program_id(0)
    T = x_ref.shape[1]
    D = x_ref.shape[2]

    sa = jax.lax.rem(g, 2) * T
    sb = jax.lax.rem(g + 1, 2) * T

    bv = bv_s[pl.ds(sb, T), :]
    rrb = rr_s[pl.ds(sb, T), :]
    a = a_s[pl.ds(sb, T), :]

    xb = x_ref[0]
    xh = xb.astype(jnp.bfloat16)
    qq = jnp.dot(xb, wq_ref[...], preferred_element_type=jnp.float32)
    kk = jnp.dot(xb, wk_ref[...], preferred_element_type=jnp.float32)
    rr = jnp.dot(xh, wres_ref[...], preferred_element_type=jnp.float32)
    rr = rr + bres_ref[...]

    row = jax.lax.broadcasted_iota(jnp.int32, (T, 1), 0)
    first_a = jax.lax.rem(g, n_per_batch) == 0
    q_carry = jnp.where(first_a, 0.0, carry_ref[1:2, :])

    q_shift = jnp.where(row == 0, q_carry, pltpu.roll(qq, 1, axis=0))

    qn2 = jnp.sum(q_shift * q_shift, axis=1, keepdims=True)
    kn2 = jnp.sum(kk * kk, axis=1, keepdims=True)
    qk = jnp.sum(q_shift * kk, axis=1, keepdims=True)
    cos = qk * jax.lax.rsqrt(jnp.maximum(qn2, 1e-16) * jnp.maximum(kn2, 1e-16))

    p_eff = jnp.where(cos <= 0.0,
                      jnp.minimum(0.5 - 0.5 * cos, 1.0 - _EPS), 0.0)
    p_eff = jnp.where(first_a & (row == 0), 1.0 - _EPS, p_eff)

    bv_s[pl.ds(sa, T), :] = p_eff.astype(jnp.bfloat16) * xh
    rr_s[pl.ds(sa, T), :] = rr.astype(jnp.bfloat16)
    a_s[pl.ds(sa, T), :] = (1.0 - p_eff).astype(jnp.bfloat16)
    carry_ref[1:2, :] = qq[T - 1:T, :]

    first_b = jax.lax.rem(g - 1, n_per_batch) == 0
    z_carry = jnp.where(first_b, 0.0, carry_ref[0:1, :])

    one = jnp.bfloat16(1.0)
    zero = jnp.bfloat16(0.0)

    d = 1
    while d < T:
        pred = row >= d
        am = jnp.where(pred, a, zero)
        bv = am * pltpu.roll(bv, d, axis=0) + bv
        a = a * jnp.where(pred, pltpu.roll(a, d, axis=0), one)
        d *= 2

    o_ref[0] = (rrb + bv).astype(jnp.float32) + a.astype(jnp.float32) * z_carry

    carry_ref[0:1, :] = (bv[T - 1:T, :].astype(jnp.float32)
                         + a[T - 1:T, :].astype(jnp.float32) * z_carry)


def kernel(x, Wq, Wk, Wres, bres):
    B, L, D = x.shape
    T = _T
    N = L // T
    grid = (B * N + 1,)
    last = B * N - 1

    def xmap(g):
        gc = jnp.minimum(g, last)
        return (gc // N, gc % N, 0)

    def omap(g):
        gc = jnp.maximum(g - 1, 0)
        return (gc // N, gc % N, 0)

    out = pl.pallas_call(
        functools.partial(_hnet_body, N),
        grid=grid,
        in_specs=[
            pl.BlockSpec((1, T, D), xmap),
            pl.BlockSpec((D, D), lambda g: (0, 0)),
            pl.BlockSpec((D, D), lambda g: (0, 0)),
            pl.BlockSpec((D, D), lambda g: (0, 0)),
            pl.BlockSpec((1, D), lambda g: (0, 0)),
        ],
        out_specs=pl.BlockSpec((1, T, D), omap),
        out_shape=jax.ShapeDtypeStruct((B, L, D), jnp.float32),
        scratch_shapes=[
            pltpu.VMEM((2 * T, D), jnp.bfloat16),
            pltpu.VMEM((2 * T, D), jnp.bfloat16),
            pltpu.VMEM((2 * T, 1), jnp.bfloat16),
            pltpu.VMEM((2, D), jnp.float32),
        ],
        compiler_params=pltpu.CompilerParams(
            dimension_semantics=("arbitrary",),
        ),
    )(x, Wq, Wk, Wres.astype(jnp.bfloat16), bres.reshape(1, D))
    return out
